# baseline (speedup 1.0000x reference)
import functools

import jax
import jax.numpy as jnp
from jax import lax
from jax.experimental import pallas as pl
from jax.experimental.pallas import tpu as pltpu
from jax.experimental.pallas import tpu_sc as plsc

_E = 8
_T = 2048
_NW = 32
_L = 16


def _logits_body(x_ref, w_ref, lg_ref):
    lg_ref[...] = jax.lax.dot_general(
        w_ref[...], x_ref[...], (((1,), (1,)), ((), ())),
        preferred_element_type=jnp.float32)


def _top2_step(lbuf, off):
    m1 = jnp.full((_L,), -jnp.inf, jnp.float32)
    m2 = jnp.full((_L,), -jnp.inf, jnp.float32)
    i1 = jnp.zeros((_L,), jnp.int32)
    i2 = jnp.zeros((_L,), jnp.int32)
    for e in range(_E):
        v = lbuf[e, pl.ds(off, _L)]
        ev = jnp.full((_L,), e, jnp.int32)
        gt1 = v > m1
        gt2 = v > m2
        i2 = jnp.where(gt1, i1, jnp.where(gt2, ev, i2))
        m2 = jnp.where(gt1, m1, jnp.where(gt2, v, m2))
        i1 = jnp.where(gt1, ev, i1)
        m1 = jnp.where(gt1, v, m1)
    w1 = 1.0 / (1.0 + jnp.exp(m2 - m1))
    return i1, i2, w1, 1.0 - w1


def _route_body(lg_hbm, i1_hbm, i2_hbm, w1_hbm, w2_hbm,
                lbuf, i1b, i2b, w1b, w2b):
    n = lg_hbm.shape[1]
    tpw = n // _NW
    wid = lax.axis_index("s") * 2 + lax.axis_index("c")
    base = wid * tpw
    pltpu.sync_copy(lg_hbm.at[:, pl.ds(base, tpw)], lbuf)

    def step(j, carry):
        off = j * _L
        i1, i2, w1, w2 = _top2_step(lbuf, off)
        i1b[pl.ds(off, _L)] = i1
        i2b[pl.ds(off, _L)] = i2
        w1b[pl.ds(off, _L)] = w1
        w2b[pl.ds(off, _L)] = w2
        return carry

    lax.fori_loop(0, tpw // _L, step, 0)
    pltpu.sync_copy(i1b, i1_hbm.at[pl.ds(base, tpw)])
    pltpu.sync_copy(i2b, i2_hbm.at[pl.ds(base, tpw)])
    pltpu.sync_copy(w1b, w1_hbm.at[pl.ds(base, tpw)])
    pltpu.sync_copy(w2b, w2_hbm.at[pl.ds(base, tpw)])


def kernel(hidden_states, weights):
    b, s, d = hidden_states.shape
    n = b * s
    hs = hidden_states.reshape(n, d)
    logits_t = pl.pallas_call(
        _logits_body,
        grid=(n // _T,),
        in_specs=[
            pl.BlockSpec((_T, d), lambda i: (i, 0)),
            pl.BlockSpec((_E, d), lambda i: (0, 0)),
        ],
        out_specs=pl.BlockSpec((_E, _T), lambda i: (0, i)),
        out_shape=jax.ShapeDtypeStruct((_E, n), jnp.float32),
    )(hs, weights)

    tpw = n // _NW
    route = functools.partial(
        pl.kernel,
        out_type=[
            jax.ShapeDtypeStruct((n,), jnp.int32),
            jax.ShapeDtypeStruct((n,), jnp.int32),
            jax.ShapeDtypeStruct((n,), jnp.float32),
            jax.ShapeDtypeStruct((n,), jnp.float32),
        ],
        mesh=plsc.VectorSubcoreMesh(core_axis_name="c", subcore_axis_name="s"),
        scratch_types=[
            pltpu.VMEM((_E, tpw), jnp.float32),
            pltpu.VMEM((tpw,), jnp.int32),
            pltpu.VMEM((tpw,), jnp.int32),
            pltpu.VMEM((tpw,), jnp.float32),
            pltpu.VMEM((tpw,), jnp.float32),
        ],
    )(_route_body)
    i1, i2, w1, w2 = route(logits_t)
    idx = jnp.stack([i1, i2], axis=-1)
    wgt = jnp.stack([w1, w2], axis=-1)
    return idx, wgt, jnp.float32(0.0)


# speedup vs baseline: 1.3657x; 1.3657x over previous
import jax
import jax.numpy as jnp
from jax.experimental import pallas as pl

_E = 8
_T = 1024


def _router_body(x_ref, w_ref, idx_ref, wgt_ref):
    x = x_ref[...]
    w = w_ref[...]
    logits = jax.lax.dot_general(
        w, x, (((1,), (1,)), ((), ())), preferred_element_type=jnp.float32)
    eidx = jax.lax.broadcasted_iota(jnp.int32, logits.shape, 0)
    m1 = jnp.max(logits, axis=0, keepdims=True)
    i1 = jnp.min(jnp.where(logits == m1, eidx, _E), axis=0, keepdims=True)
    masked = jnp.where(eidx == i1, -jnp.inf, logits)
    m2 = jnp.max(masked, axis=0, keepdims=True)
    i2 = jnp.min(jnp.where(masked == m2, eidx, _E), axis=0, keepdims=True)
    w1 = 1.0 / (1.0 + jnp.exp(m2 - m1))
    idx_ref[...] = jnp.concatenate([i1, i2], axis=0)
    wgt_ref[...] = jnp.concatenate([w1, 1.0 - w1], axis=0)


def kernel(hidden_states, weights):
    b, s, d = hidden_states.shape
    n = b * s
    hs = hidden_states.reshape(n, d)
    idx_t, wgt_t = pl.pallas_call(
        _router_body,
        grid=(n // _T,),
        in_specs=[
            pl.BlockSpec((_T, d), lambda i: (i, 0)),
            pl.BlockSpec((_E, d), lambda i: (0, 0)),
        ],
        out_specs=[
            pl.BlockSpec((2, _T), lambda i: (0, i)),
            pl.BlockSpec((2, _T), lambda i: (0, i)),
        ],
        out_shape=[
            jax.ShapeDtypeStruct((2, n), jnp.int32),
            jax.ShapeDtypeStruct((2, n), jnp.float32),
        ],
    )(hs, weights)
    return idx_t.T, wgt_t.T, jnp.float32(0.0)


# speedup vs baseline: 1.7486x; 1.2804x over previous
import jax
import jax.numpy as jnp
from jax.experimental import pallas as pl

_E = 8
_T = 4096


def _router_body(x_ref, w_ref, idx_ref, wgt_ref):
    x = x_ref[...]
    w = w_ref[...]
    logits = jax.lax.dot_general(
        w, x, (((1,), (1,)), ((), ())), preferred_element_type=jnp.float32)
    eidx = jax.lax.broadcasted_iota(jnp.int32, logits.shape, 0)
    m1 = jnp.max(logits, axis=0, keepdims=True)
    i1 = jnp.min(jnp.where(logits == m1, eidx, _E), axis=0, keepdims=True)
    masked = jnp.where(eidx == i1, -jnp.inf, logits)
    m2 = jnp.max(masked, axis=0, keepdims=True)
    i2 = jnp.min(jnp.where(masked == m2, eidx, _E), axis=0, keepdims=True)
    w1 = 1.0 / (1.0 + jnp.exp(m2 - m1))
    idx_ref[...] = jnp.concatenate([i1, i2], axis=0)
    wgt_ref[...] = jnp.concatenate([w1, 1.0 - w1], axis=0)


def kernel(hidden_states, weights):
    b, s, d = hidden_states.shape
    n = b * s
    hs = hidden_states.reshape(n, d)
    idx_t, wgt_t = pl.pallas_call(
        _router_body,
        grid=(n // _T,),
        in_specs=[
            pl.BlockSpec((_T, d), lambda i: (i, 0)),
            pl.BlockSpec((_E, d), lambda i: (0, 0)),
        ],
        out_specs=[
            pl.BlockSpec((2, _T), lambda i: (0, i)),
            pl.BlockSpec((2, _T), lambda i: (0, i)),
        ],
        out_shape=[
            jax.ShapeDtypeStruct((2, n), jnp.int32),
            jax.ShapeDtypeStruct((2, n), jnp.float32),
        ],
    )(hs, weights)
    return idx_t.T, wgt_t.T, jnp.float32(0.0)
